# baseline (speedup 1.0000x reference)
import functools

import jax
import jax.numpy as jnp
from jax import lax
from jax.experimental import pallas as pl
from jax.experimental.pallas import tpu as pltpu
from jax.experimental.pallas import tpu_sc as plsc

D = 64
L = 16
NW = 32
BPW = 512
C = 32
IDXPAD = 528


def _take(v, idx):
    return jnp.take_along_axis(v, idx, axis=0, mode="promise_in_bounds")


def _wid():
    return lax.axis_index("s") * 2 + lax.axis_index("c")


def _gather_body(cst_u, rst_u, lan_u, nu_u, cst_i, rst_i, lan_i, nu_i,
                 ufacT, ifacT, stg_u, stg_i,
                 col_v, rst_v, lan_v, nu_v, slab_v, stage_v, sem):
    wid = _wid()
    lane = lax.iota(jnp.int32, L)
    dvec = [lane + q * L for q in range(D // L)]

    for cst_h, rst_h, lan_h, nu_h, tbl, stg in (
            (cst_u, rst_u, lan_u, nu_u, ufacT, stg_u),
            (cst_i, rst_i, lan_i, nu_i, ifacT, stg_i)):
        pltpu.sync_copy(cst_h.at[wid], col_v)
        pltpu.sync_copy(rst_h.at[wid], rst_v)
        pltpu.sync_copy(lan_h.at[wid], lan_v)
        pltpu.sync_copy(nu_h, nu_v)
        nvec = nu_v[pl.ds((wid >> 4) << 4, L)]
        n = jnp.sum(jnp.where(lane == (wid & 15), nvec, 0))

        c0 = pl.multiple_of(col_v[pl.ds(0, L)][0], 128)
        pltpu.async_copy(tbl.at[:, pl.ds(c0, 128)], slab_v.at[0], sem.at[0])

        def slab_body(s, carry, tbl=tbl):
            @pl.when(s + 1 < n)
            def _start_next():
                cn = pl.multiple_of(col_v[pl.ds(s + 1, L)][0], 128)
                pltpu.async_copy(tbl.at[:, pl.ds(cn, 128)],
                                 slab_v.at[(s + 1) & 1], sem.at[(s + 1) & 1])

            pltpu.make_async_copy(tbl.at[:, pl.ds(0, 128)],
                                  slab_v.at[s & 1], sem.at[s & 1]).wait()
            rvec = rst_v[pl.ds(s, L)]
            par = jnp.full((L,), s & 1, jnp.int32)

            def row_body(j, c2):
                l = lan_v[pl.ds(j, L)][0]
                lsp = jnp.full((L,), l, jnp.int32)
                for q in range(D // L):
                    g = plsc.load_gather(slab_v, [par, dvec[q], lsp])
                    stage_v[j >> 1, pl.ds((j & 1) * D + q * L, L)] = g
                return c2

            lax.fori_loop(rvec[0], rvec[1], row_body, 0)
            return carry

        lax.fori_loop(0, n, slab_body, 0)
        pltpu.sync_copy(stage_v, stg.at[pl.ds(wid * (BPW // 2), BPW // 2)])


def _pair_body(rk_u, rk_i, stg_u, stg_i, out_hbm,
               rku_v, rki_v, ub_v, ib_v, out_v, sem):
    wid = _wid()
    base = wid * BPW
    pltpu.sync_copy(rk_u.at[wid], rku_v)
    pltpu.sync_copy(rk_i.at[wid], rki_v)

    lane = lax.iota(jnp.int32, L)
    xor_idx = [lane ^ sh for sh in (8, 4, 2, 1)]
    zero = jnp.zeros((L,), jnp.float32)

    def chunk_body(k, carry):
        copies = []
        rvecs = []
        for blk in range(C // L):
            ruv = rku_v[pl.ds(k * C + blk * L, L)]
            riv = rki_v[pl.ds(k * C + blk * L, L)]
            rvecs.append((ruv, riv))
            for j in range(L):
                slot = blk * L + j
                copies.append(pltpu.async_copy(
                    stg_u.at[ruv[j] >> 1], ub_v.at[slot], sem))
                copies.append(pltpu.async_copy(
                    stg_i.at[riv[j] >> 1], ib_v.at[slot], sem))
        for cp in copies:
            cp.wait()

        for blk in range(C // L):
            ruv, riv = rvecs[blk]
            acc = zero
            for j in range(L):
                slot = blk * L + j
                ou = (ruv[j] & 1) * D
                oi = (riv[j] & 1) * D
                p = ub_v[slot, pl.ds(ou, L)] * ib_v[slot, pl.ds(oi, L)]
                for q in range(1, D // L):
                    p += (ub_v[slot, pl.ds(ou + q * L, L)]
                          * ib_v[slot, pl.ds(oi + q * L, L)])
                for xi in xor_idx:
                    p = p + _take(p, xi)
                acc = jnp.where(lane == j, p, acc)
            out_v[pl.ds(k * C + blk * L, L)] = acc
        return carry

    lax.fori_loop(0, BPW // C, chunk_body, 0)
    pltpu.sync_copy(out_v, out_hbm.at[pl.ds(base, BPW)])


def _side_routing(idx):
    order = jnp.argsort(idx)
    xs = idx[order]
    cstart = (xs >> 7) << 7
    lanes = xs & 127
    rank = jnp.argsort(order).astype(jnp.int32)
    c2 = cstart.reshape(NW, BPW)
    newflag = jnp.concatenate(
        [jnp.ones((NW, 1), jnp.bool_), c2[:, 1:] != c2[:, :-1]], axis=1)
    ord2 = jnp.cumsum(newflag.astype(jnp.int32), axis=1) - 1
    nu = (ord2[:, -1] + 1).astype(jnp.int32)
    warange = jnp.arange(NW, dtype=jnp.int32)[:, None]
    cst = jnp.zeros((NW, IDXPAD), jnp.int32).at[warange, ord2].set(c2)
    jrange = jnp.broadcast_to(
        jnp.arange(BPW, dtype=jnp.int32)[None, :], (NW, BPW))
    rst = jnp.full((NW, IDXPAD), BPW, jnp.int32).at[warange, ord2].min(jrange)
    lan = jnp.zeros((NW, IDXPAD), jnp.int32).at[
        warange, jrange].set(lanes.reshape(NW, BPW))
    return cst, rst, lan, nu, rank.reshape(NW, BPW)


def kernel(user_item_tuple, user_factors, item_factors):
    batch = user_item_tuple.shape[0]
    uit = user_item_tuple.astype(jnp.int32)
    cst_u, rst_u, lan_u, nu_u, rk_u = _side_routing(uit[:, 0])
    cst_i, rst_i, lan_i, nu_i, rk_i = _side_routing(uit[:, 1])

    mesh = plsc.VectorSubcoreMesh(core_axis_name="c", subcore_axis_name="s")
    params = pltpu.CompilerParams(needs_layout_passes=False)
    stg_shape = jax.ShapeDtypeStruct((batch // 2, 2 * D), jnp.float32)

    gather_call = functools.partial(
        pl.kernel,
        out_type=(stg_shape, stg_shape),
        mesh=mesh,
        compiler_params=params,
        scratch_types=[
            pltpu.VMEM((IDXPAD,), jnp.int32),
            pltpu.VMEM((IDXPAD,), jnp.int32),
            pltpu.VMEM((IDXPAD,), jnp.int32),
            pltpu.VMEM((NW,), jnp.int32),
            pltpu.VMEM((2, D, 128), jnp.float32),
            pltpu.VMEM((BPW // 2, 2 * D), jnp.float32),
            pltpu.SemaphoreType.DMA((2,)),
        ],
    )(_gather_body)
    stg_u, stg_i = gather_call(
        cst_u, rst_u, lan_u, nu_u, cst_i, rst_i, lan_i, nu_i,
        user_factors.T, item_factors.T)

    pair_call = functools.partial(
        pl.kernel,
        out_type=jax.ShapeDtypeStruct((batch,), jnp.float32),
        mesh=mesh,
        compiler_params=params,
        scratch_types=[
            pltpu.VMEM((BPW,), jnp.int32),
            pltpu.VMEM((BPW,), jnp.int32),
            pltpu.VMEM((C, 2 * D), jnp.float32),
            pltpu.VMEM((C, 2 * D), jnp.float32),
            pltpu.VMEM((BPW,), jnp.float32),
            pltpu.SemaphoreType.DMA,
        ],
    )(_pair_body)
    return pair_call(rk_u, rk_i, stg_u, stg_i)


# speedup vs baseline: 2.1614x; 2.1614x over previous
import functools

import jax
import jax.numpy as jnp
from jax import lax
from jax.experimental import pallas as pl
from jax.experimental.pallas import tpu as pltpu
from jax.experimental.pallas import tpu_sc as plsc

D = 64
L = 16
NW = 32
BPW = 512
C = 32
NBUF = 4
IDXPAD = 528


def _take(v, idx):
    return jnp.take_along_axis(v, idx, axis=0, mode="promise_in_bounds")


def _wid():
    return lax.axis_index("s") * 2 + lax.axis_index("c")


def _gather_body(cst_u, rst_u, lan_u, nu_u, cst_i, rst_i, lan_i, nu_i,
                 ufacT, ifacT, stg_u, stg_i,
                 col_v, rst_v, lan_v, nu_v, slab_v, stage_v, sem):
    wid = _wid()
    lane = lax.iota(jnp.int32, L)
    dvec = [lane + q * L for q in range(D // L)]

    for cst_h, rst_h, lan_h, nu_h, tbl, stg in (
            (cst_u, rst_u, lan_u, nu_u, ufacT, stg_u),
            (cst_i, rst_i, lan_i, nu_i, ifacT, stg_i)):
        pltpu.sync_copy(cst_h.at[wid], col_v)
        pltpu.sync_copy(rst_h.at[wid], rst_v)
        pltpu.sync_copy(lan_h.at[wid], lan_v)
        pltpu.sync_copy(nu_h, nu_v)
        nvec = nu_v[pl.ds((wid >> 4) << 4, L)]
        n = jnp.sum(jnp.where(lane == (wid & 15), nvec, 0))

        for pre in range(NBUF - 1):
            @pl.when(pre < n)
            def _prime(pre=pre):
                cp = pl.multiple_of(col_v[pl.ds(pre, L)][0], 128)
                pltpu.async_copy(tbl.at[:, pl.ds(cp, 128)],
                                 slab_v.at[pre], sem.at[pre])

        def slab_body(s, carry, tbl=tbl):
            @pl.when(s + NBUF - 1 < n)
            def _start_next():
                cn = pl.multiple_of(col_v[pl.ds(s + NBUF - 1, L)][0], 128)
                pltpu.async_copy(
                    tbl.at[:, pl.ds(cn, 128)],
                    slab_v.at[(s + NBUF - 1) & (NBUF - 1)],
                    sem.at[(s + NBUF - 1) & (NBUF - 1)])

            pltpu.make_async_copy(tbl.at[:, pl.ds(0, 128)],
                                  slab_v.at[s & (NBUF - 1)],
                                  sem.at[s & (NBUF - 1)]).wait()
            rvec = rst_v[pl.ds(s, L)]
            par = jnp.full((L,), s & (NBUF - 1), jnp.int32)

            def row_body(j, c2):
                l = lan_v[pl.ds(j, L)][0]
                lsp = jnp.full((L,), l, jnp.int32)
                for q in range(D // L):
                    g = plsc.load_gather(slab_v, [par, dvec[q], lsp])
                    stage_v[j >> 1, pl.ds((j & 1) * D + q * L, L)] = g
                return c2

            lax.fori_loop(rvec[0], rvec[1], row_body, 0)
            return carry

        lax.fori_loop(0, n, slab_body, 0)
        pltpu.sync_copy(stage_v, stg.at[pl.ds(wid * (BPW // 2), BPW // 2)])


def _pair_body(rk_u, rk_i, stg_u, stg_i, out_hbm,
               rku_v, rki_v, ub_v, ib_v, out_v, sem):
    wid = _wid()
    base = wid * BPW
    pltpu.sync_copy(rk_u.at[wid], rku_v)
    pltpu.sync_copy(rk_i.at[wid], rki_v)

    lane = lax.iota(jnp.int32, L)
    xor_idx = [lane ^ sh for sh in (8, 4, 2, 1)]
    zero = jnp.zeros((L,), jnp.float32)

    def chunk_body(k, carry):
        copies = []
        rvecs = []
        for blk in range(C // L):
            ruv = rku_v[pl.ds(k * C + blk * L, L)]
            riv = rki_v[pl.ds(k * C + blk * L, L)]
            rvecs.append((ruv, riv))
            for j in range(L):
                slot = blk * L + j
                copies.append(pltpu.async_copy(
                    stg_u.at[ruv[j] >> 1], ub_v.at[slot], sem))
                copies.append(pltpu.async_copy(
                    stg_i.at[riv[j] >> 1], ib_v.at[slot], sem))
        for cp in copies:
            cp.wait()

        for blk in range(C // L):
            ruv, riv = rvecs[blk]
            acc = zero
            for j in range(L):
                slot = blk * L + j
                ou = (ruv[j] & 1) * D
                oi = (riv[j] & 1) * D
                p = ub_v[slot, pl.ds(ou, L)] * ib_v[slot, pl.ds(oi, L)]
                for q in range(1, D // L):
                    p += (ub_v[slot, pl.ds(ou + q * L, L)]
                          * ib_v[slot, pl.ds(oi + q * L, L)])
                for xi in xor_idx:
                    p = p + _take(p, xi)
                acc = jnp.where(lane == j, p, acc)
            out_v[pl.ds(k * C + blk * L, L)] = acc
        return carry

    lax.fori_loop(0, BPW // C, chunk_body, 0)
    pltpu.sync_copy(out_v, out_hbm.at[pl.ds(base, BPW)])


def _side_routing(idx):
    order = jnp.argsort(idx)
    xs = idx[order]
    cstart = (xs >> 7) << 7
    lanes = xs & 127
    rank = jnp.argsort(order).astype(jnp.int32)
    c2 = cstart.reshape(NW, BPW)
    newflag = jnp.concatenate(
        [jnp.ones((NW, 1), jnp.bool_), c2[:, 1:] != c2[:, :-1]], axis=1)
    ord2 = jnp.cumsum(newflag.astype(jnp.int32), axis=1) - 1
    nu = (ord2[:, -1] + 1).astype(jnp.int32)
    karange = jnp.arange(IDXPAD, dtype=jnp.int32)[None, None, :]
    rst = (ord2[:, :, None] < karange).astype(jnp.int32).sum(axis=1)
    cst = jnp.take_along_axis(c2, jnp.minimum(rst, BPW - 1), axis=1,
                              mode="promise_in_bounds")
    lan = jnp.pad(lanes.reshape(NW, BPW), ((0, 0), (0, IDXPAD - BPW)))
    return cst, rst, lan, nu, rank.reshape(NW, BPW)


def kernel(user_item_tuple, user_factors, item_factors):
    batch = user_item_tuple.shape[0]
    uit = user_item_tuple.astype(jnp.int32)
    cst_u, rst_u, lan_u, nu_u, rk_u = _side_routing(uit[:, 0])
    cst_i, rst_i, lan_i, nu_i, rk_i = _side_routing(uit[:, 1])

    mesh = plsc.VectorSubcoreMesh(core_axis_name="c", subcore_axis_name="s")
    params = pltpu.CompilerParams(needs_layout_passes=False)
    stg_shape = jax.ShapeDtypeStruct((batch // 2, 2 * D), jnp.float32)

    gather_call = functools.partial(
        pl.kernel,
        out_type=(stg_shape, stg_shape),
        mesh=mesh,
        compiler_params=params,
        scratch_types=[
            pltpu.VMEM((IDXPAD,), jnp.int32),
            pltpu.VMEM((IDXPAD,), jnp.int32),
            pltpu.VMEM((IDXPAD,), jnp.int32),
            pltpu.VMEM((NW,), jnp.int32),
            pltpu.VMEM((NBUF, D, 128), jnp.float32),
            pltpu.VMEM((BPW // 2, 2 * D), jnp.float32),
            pltpu.SemaphoreType.DMA((NBUF,)),
        ],
    )(_gather_body)
    stg_u, stg_i = gather_call(
        cst_u, rst_u, lan_u, nu_u, cst_i, rst_i, lan_i, nu_i,
        user_factors.T, item_factors.T)

    pair_call = functools.partial(
        pl.kernel,
        out_type=jax.ShapeDtypeStruct((batch,), jnp.float32),
        mesh=mesh,
        compiler_params=params,
        scratch_types=[
            pltpu.VMEM((BPW,), jnp.int32),
            pltpu.VMEM((BPW,), jnp.int32),
            pltpu.VMEM((C, 2 * D), jnp.float32),
            pltpu.VMEM((C, 2 * D), jnp.float32),
            pltpu.VMEM((BPW,), jnp.float32),
            pltpu.SemaphoreType.DMA,
        ],
    )(_pair_body)
    return pair_call(rk_u, rk_i, stg_u, stg_i)


# speedup vs baseline: 2.3503x; 1.0874x over previous
import functools

import jax
import jax.numpy as jnp
from jax import lax
from jax.experimental import pallas as pl
from jax.experimental.pallas import tpu as pltpu
from jax.experimental.pallas import tpu_sc as plsc

D = 64
L = 16
NW = 32
BPW = 512
C = 32
NBUF = 8
IDXPAD = 528


def _take(v, idx):
    return jnp.take_along_axis(v, idx, axis=0, mode="promise_in_bounds")


def _wid():
    return lax.axis_index("s") * 2 + lax.axis_index("c")


def _gather_body(cst_u, rst_u, lan_u, nu_u, cst_i, rst_i, lan_i, nu_i,
                 ufacT, ifacT, stg_u, stg_i,
                 col_v, rst_v, lan_v, nu_v, slab_v, stage_v, sem):
    wid = _wid()
    lane = lax.iota(jnp.int32, L)
    dvec = [lane + q * L for q in range(D // L)]

    for cst_h, rst_h, lan_h, nu_h, tbl, stg in (
            (cst_u, rst_u, lan_u, nu_u, ufacT, stg_u),
            (cst_i, rst_i, lan_i, nu_i, ifacT, stg_i)):
        pltpu.sync_copy(cst_h.at[wid], col_v)
        pltpu.sync_copy(rst_h.at[wid], rst_v)
        pltpu.sync_copy(lan_h.at[wid], lan_v)
        pltpu.sync_copy(nu_h, nu_v)
        nvec = nu_v[pl.ds((wid >> 4) << 4, L)]
        n = jnp.sum(jnp.where(lane == (wid & 15), nvec, 0))

        for pre in range(NBUF - 1):
            @pl.when(pre < n)
            def _prime(pre=pre):
                cp = pl.multiple_of(col_v[pl.ds(pre, L)][0], 128)
                pltpu.async_copy(tbl.at[:, pl.ds(cp, 128)],
                                 slab_v.at[pre], sem.at[pre])

        def slab_body(s, carry, tbl=tbl):
            @pl.when(s + NBUF - 1 < n)
            def _start_next():
                cn = pl.multiple_of(col_v[pl.ds(s + NBUF - 1, L)][0], 128)
                pltpu.async_copy(
                    tbl.at[:, pl.ds(cn, 128)],
                    slab_v.at[(s + NBUF - 1) & (NBUF - 1)],
                    sem.at[(s + NBUF - 1) & (NBUF - 1)])

            pltpu.make_async_copy(tbl.at[:, pl.ds(0, 128)],
                                  slab_v.at[s & (NBUF - 1)],
                                  sem.at[s & (NBUF - 1)]).wait()
            rvec = rst_v[pl.ds(s, L)]
            par = jnp.full((L,), s & (NBUF - 1), jnp.int32)

            def row_body(j, c2):
                l = lan_v[pl.ds(j, L)][0]
                lsp = jnp.full((L,), l, jnp.int32)
                for q in range(D // L):
                    g = plsc.load_gather(slab_v, [par, dvec[q], lsp])
                    stage_v[j >> 1, pl.ds((j & 1) * D + q * L, L)] = g
                return c2

            lax.fori_loop(rvec[0], rvec[1], row_body, 0)
            return carry

        lax.fori_loop(0, n, slab_body, 0)
        pltpu.sync_copy(stage_v, stg.at[pl.ds(wid * (BPW // 2), BPW // 2)])


def _pair_body(rk_u, rk_i, stg_u, stg_i, out_hbm,
               rku_v, rki_v, ub_v, ib_v, out_v, sem):
    wid = _wid()
    base = wid * BPW
    pltpu.sync_copy(rk_u.at[wid], rku_v)
    pltpu.sync_copy(rk_i.at[wid], rki_v)

    lane = lax.iota(jnp.int32, L)
    xor_idx = [lane ^ sh for sh in (8, 4, 2, 1)]
    zero = jnp.zeros((L,), jnp.float32)

    def chunk_body(k, carry):
        copies = []
        rvecs = []
        for blk in range(C // L):
            ruv = rku_v[pl.ds(k * C + blk * L, L)]
            riv = rki_v[pl.ds(k * C + blk * L, L)]
            rvecs.append((ruv, riv))
            for j in range(L):
                slot = blk * L + j
                copies.append(pltpu.async_copy(
                    stg_u.at[ruv[j] >> 1], ub_v.at[slot], sem))
                copies.append(pltpu.async_copy(
                    stg_i.at[riv[j] >> 1], ib_v.at[slot], sem))
        for cp in copies:
            cp.wait()

        for blk in range(C // L):
            ruv, riv = rvecs[blk]
            acc = zero
            for j in range(L):
                slot = blk * L + j
                ou = (ruv[j] & 1) * D
                oi = (riv[j] & 1) * D
                p = ub_v[slot, pl.ds(ou, L)] * ib_v[slot, pl.ds(oi, L)]
                for q in range(1, D // L):
                    p += (ub_v[slot, pl.ds(ou + q * L, L)]
                          * ib_v[slot, pl.ds(oi + q * L, L)])
                for xi in xor_idx:
                    p = p + _take(p, xi)
                acc = jnp.where(lane == j, p, acc)
            out_v[pl.ds(k * C + blk * L, L)] = acc
        return carry

    lax.fori_loop(0, BPW // C, chunk_body, 0)
    pltpu.sync_copy(out_v, out_hbm.at[pl.ds(base, BPW)])


def _side_routing(idx):
    order = jnp.argsort(idx)
    xs = idx[order]
    cstart = (xs >> 7) << 7
    lanes = xs & 127
    rank = jnp.argsort(order).astype(jnp.int32)
    c2 = cstart.reshape(NW, BPW)
    newflag = jnp.concatenate(
        [jnp.ones((NW, 1), jnp.bool_), c2[:, 1:] != c2[:, :-1]], axis=1)
    ord2 = jnp.cumsum(newflag.astype(jnp.int32), axis=1) - 1
    nu = (ord2[:, -1] + 1).astype(jnp.int32)
    karange = jnp.arange(IDXPAD, dtype=jnp.int32)[None, None, :]
    rst = (ord2[:, :, None] < karange).astype(jnp.int32).sum(axis=1)
    cst = jnp.take_along_axis(c2, jnp.minimum(rst, BPW - 1), axis=1,
                              mode="promise_in_bounds")
    lan = jnp.pad(lanes.reshape(NW, BPW), ((0, 0), (0, IDXPAD - BPW)))
    return cst, rst, lan, nu, rank.reshape(NW, BPW)


def kernel(user_item_tuple, user_factors, item_factors):
    batch = user_item_tuple.shape[0]
    uit = user_item_tuple.astype(jnp.int32)
    cst_u, rst_u, lan_u, nu_u, rk_u = _side_routing(uit[:, 0])
    cst_i, rst_i, lan_i, nu_i, rk_i = _side_routing(uit[:, 1])

    mesh = plsc.VectorSubcoreMesh(core_axis_name="c", subcore_axis_name="s")
    params = pltpu.CompilerParams(needs_layout_passes=False)
    stg_shape = jax.ShapeDtypeStruct((batch // 2, 2 * D), jnp.float32)

    gather_call = functools.partial(
        pl.kernel,
        out_type=(stg_shape, stg_shape),
        mesh=mesh,
        compiler_params=params,
        scratch_types=[
            pltpu.VMEM((IDXPAD,), jnp.int32),
            pltpu.VMEM((IDXPAD,), jnp.int32),
            pltpu.VMEM((IDXPAD,), jnp.int32),
            pltpu.VMEM((NW,), jnp.int32),
            pltpu.VMEM((NBUF, D, 128), jnp.float32),
            pltpu.VMEM((BPW // 2, 2 * D), jnp.float32),
            pltpu.SemaphoreType.DMA((NBUF,)),
        ],
    )(_gather_body)
    stg_u, stg_i = gather_call(
        cst_u, rst_u, lan_u, nu_u, cst_i, rst_i, lan_i, nu_i,
        user_factors.T, item_factors.T)

    pair_call = functools.partial(
        pl.kernel,
        out_type=jax.ShapeDtypeStruct((batch,), jnp.float32),
        mesh=mesh,
        compiler_params=params,
        scratch_types=[
            pltpu.VMEM((BPW,), jnp.int32),
            pltpu.VMEM((BPW,), jnp.int32),
            pltpu.VMEM((C, 2 * D), jnp.float32),
            pltpu.VMEM((C, 2 * D), jnp.float32),
            pltpu.VMEM((BPW,), jnp.float32),
            pltpu.SemaphoreType.DMA,
        ],
    )(_pair_body)
    return pair_call(rk_u, rk_i, stg_u, stg_i)


# speedup vs baseline: 2.4464x; 1.0409x over previous
import functools

import jax
import jax.numpy as jnp
from jax import lax
from jax.experimental import pallas as pl
from jax.experimental.pallas import tpu as pltpu
from jax.experimental.pallas import tpu_sc as plsc

D = 64
L = 16
NW = 32
BPW = 512
C = 64
NBUF = 8
IDXPAD = 528


def _take(v, idx):
    return jnp.take_along_axis(v, idx, axis=0, mode="promise_in_bounds")


def _wid():
    return lax.axis_index("s") * 2 + lax.axis_index("c")


def _gather_body(cst_u, rst_u, lan_u, nu_u, cst_i, rst_i, lan_i, nu_i,
                 ufacT, ifacT, stg_u, stg_i,
                 col_v, rst_v, lan_v, nu_v, slab_v, stage_v, sem):
    wid = _wid()
    lane = lax.iota(jnp.int32, L)
    dvec = [lane + q * L for q in range(D // L)]

    for cst_h, rst_h, lan_h, nu_h, tbl, stg in (
            (cst_u, rst_u, lan_u, nu_u, ufacT, stg_u),
            (cst_i, rst_i, lan_i, nu_i, ifacT, stg_i)):
        pltpu.sync_copy(cst_h.at[wid], col_v)
        pltpu.sync_copy(rst_h.at[wid], rst_v)
        pltpu.sync_copy(lan_h.at[wid], lan_v)
        pltpu.sync_copy(nu_h, nu_v)
        nvec = nu_v[pl.ds((wid >> 4) << 4, L)]
        n = jnp.sum(jnp.where(lane == (wid & 15), nvec, 0))

        for pre in range(NBUF - 1):
            @pl.when(pre < n)
            def _prime(pre=pre):
                cp = pl.multiple_of(col_v[pl.ds(pre, L)][0], 128)
                pltpu.async_copy(tbl.at[:, pl.ds(cp, 128)],
                                 slab_v.at[pre], sem.at[pre])

        def slab_body(s, carry, tbl=tbl):
            @pl.when(s + NBUF - 1 < n)
            def _start_next():
                cn = pl.multiple_of(col_v[pl.ds(s + NBUF - 1, L)][0], 128)
                pltpu.async_copy(
                    tbl.at[:, pl.ds(cn, 128)],
                    slab_v.at[(s + NBUF - 1) & (NBUF - 1)],
                    sem.at[(s + NBUF - 1) & (NBUF - 1)])

            pltpu.make_async_copy(tbl.at[:, pl.ds(0, 128)],
                                  slab_v.at[s & (NBUF - 1)],
                                  sem.at[s & (NBUF - 1)]).wait()
            rvec = rst_v[pl.ds(s, L)]
            par = jnp.full((L,), s & (NBUF - 1), jnp.int32)

            def row_body(j, c2):
                l = lan_v[pl.ds(j, L)][0]
                lsp = jnp.full((L,), l, jnp.int32)
                for q in range(D // L):
                    g = plsc.load_gather(slab_v, [par, dvec[q], lsp])
                    stage_v[j >> 1, pl.ds((j & 1) * D + q * L, L)] = g
                return c2

            lax.fori_loop(rvec[0], rvec[1], row_body, 0)
            return carry

        lax.fori_loop(0, n, slab_body, 0)
        pltpu.sync_copy(stage_v, stg.at[pl.ds(wid * (BPW // 2), BPW // 2)])


def _pair_body(rk_u, rk_i, stg_u, stg_i, out_hbm,
               rku_v, rki_v, hu_v, hi_v, ub_v, ib_v, out_v, sem):
    wid = _wid()
    base = wid * BPW
    pltpu.sync_copy(rk_u.at[wid], rku_v)
    pltpu.sync_copy(rk_i.at[wid], rki_v)

    lane = lax.iota(jnp.int32, L)
    xor_idx = [lane ^ sh for sh in (8, 4, 2, 1)]
    zero = jnp.zeros((L,), jnp.float32)

    for blk in range(BPW // L):
        hu_v[pl.ds(blk * L, L)] = rku_v[pl.ds(blk * L, L)] >> 1
        hi_v[pl.ds(blk * L, L)] = rki_v[pl.ds(blk * L, L)] >> 1

    def _start(k):
        pltpu.async_copy(stg_u.at[hu_v.at[pl.ds(k * C, C)]],
                         ub_v.at[k & 1], sem.at[k & 1])
        pltpu.async_copy(stg_i.at[hi_v.at[pl.ds(k * C, C)]],
                         ib_v.at[k & 1], sem.at[k & 1])

    _start(0)

    nchunks = BPW // C

    def chunk_body(k, carry):
        @pl.when(k + 1 < nchunks)
        def _next():
            _start(k + 1)
        pltpu.make_async_copy(stg_u.at[pl.ds(0, C)], ub_v.at[k & 1],
                              sem.at[k & 1]).wait()
        pltpu.make_async_copy(stg_i.at[pl.ds(0, C)], ib_v.at[k & 1],
                              sem.at[k & 1]).wait()

        for blk in range(C // L):
            ruv = rku_v[pl.ds(k * C + blk * L, L)]
            riv = rki_v[pl.ds(k * C + blk * L, L)]
            acc = zero
            for j in range(L):
                slot = blk * L + j
                ou = (ruv[j] & 1) * D
                oi = (riv[j] & 1) * D
                p = (ub_v[k & 1, slot, pl.ds(ou, L)]
                     * ib_v[k & 1, slot, pl.ds(oi, L)])
                for q in range(1, D // L):
                    p += (ub_v[k & 1, slot, pl.ds(ou + q * L, L)]
                          * ib_v[k & 1, slot, pl.ds(oi + q * L, L)])
                for xi in xor_idx:
                    p = p + _take(p, xi)
                acc = jnp.where(lane == j, p, acc)
            out_v[pl.ds(k * C + blk * L, L)] = acc
        return carry

    lax.fori_loop(0, nchunks, chunk_body, 0)
    pltpu.sync_copy(out_v, out_hbm.at[pl.ds(base, BPW)])


def _side_routing(idx):
    order = jnp.argsort(idx)
    xs = idx[order]
    cstart = (xs >> 7) << 7
    lanes = xs & 127
    rank = jnp.argsort(order).astype(jnp.int32)
    c2 = cstart.reshape(NW, BPW)
    newflag = jnp.concatenate(
        [jnp.ones((NW, 1), jnp.bool_), c2[:, 1:] != c2[:, :-1]], axis=1)
    ord2 = jnp.cumsum(newflag.astype(jnp.int32), axis=1) - 1
    nu = (ord2[:, -1] + 1).astype(jnp.int32)
    karange = jnp.arange(IDXPAD, dtype=jnp.int32)[None, None, :]
    rst = (ord2[:, :, None] < karange).astype(jnp.int32).sum(axis=1)
    cst = jnp.take_along_axis(c2, jnp.minimum(rst, BPW - 1), axis=1,
                              mode="promise_in_bounds")
    lan = jnp.pad(lanes.reshape(NW, BPW), ((0, 0), (0, IDXPAD - BPW)))
    return cst, rst, lan, nu, rank.reshape(NW, BPW)


def kernel(user_item_tuple, user_factors, item_factors):
    batch = user_item_tuple.shape[0]
    uit = user_item_tuple.astype(jnp.int32)
    cst_u, rst_u, lan_u, nu_u, rk_u = _side_routing(uit[:, 0])
    cst_i, rst_i, lan_i, nu_i, rk_i = _side_routing(uit[:, 1])

    mesh = plsc.VectorSubcoreMesh(core_axis_name="c", subcore_axis_name="s")
    params = pltpu.CompilerParams(needs_layout_passes=False)
    stg_shape = jax.ShapeDtypeStruct((batch // 2, 2 * D), jnp.float32)

    gather_call = functools.partial(
        pl.kernel,
        out_type=(stg_shape, stg_shape),
        mesh=mesh,
        compiler_params=params,
        scratch_types=[
            pltpu.VMEM((IDXPAD,), jnp.int32),
            pltpu.VMEM((IDXPAD,), jnp.int32),
            pltpu.VMEM((IDXPAD,), jnp.int32),
            pltpu.VMEM((NW,), jnp.int32),
            pltpu.VMEM((NBUF, D, 128), jnp.float32),
            pltpu.VMEM((BPW // 2, 2 * D), jnp.float32),
            pltpu.SemaphoreType.DMA((NBUF,)),
        ],
    )(_gather_body)
    stg_u, stg_i = gather_call(
        cst_u, rst_u, lan_u, nu_u, cst_i, rst_i, lan_i, nu_i,
        user_factors.T, item_factors.T)

    pair_call = functools.partial(
        pl.kernel,
        out_type=jax.ShapeDtypeStruct((batch,), jnp.float32),
        mesh=mesh,
        compiler_params=params,
        scratch_types=[
            pltpu.VMEM((BPW,), jnp.int32),
            pltpu.VMEM((BPW,), jnp.int32),
            pltpu.VMEM((BPW,), jnp.int32),
            pltpu.VMEM((BPW,), jnp.int32),
            pltpu.VMEM((2, C, 2 * D), jnp.float32),
            pltpu.VMEM((2, C, 2 * D), jnp.float32),
            pltpu.VMEM((BPW,), jnp.float32),
            pltpu.SemaphoreType.DMA((2,)),
        ],
    )(_pair_body)
    return pair_call(rk_u, rk_i, stg_u, stg_i)


# speedup vs baseline: 2.5628x; 1.0476x over previous
import functools

import jax
import jax.numpy as jnp
from jax import lax
from jax.experimental import pallas as pl
from jax.experimental.pallas import tpu as pltpu
from jax.experimental.pallas import tpu_sc as plsc

D = 64
L = 16
NW = 32
BPW = 512
C = 64
NBUF = 8
IDXPAD = 528


def _take(v, idx):
    return jnp.take_along_axis(v, idx, axis=0, mode="promise_in_bounds")


def _wid():
    return lax.axis_index("s") * 2 + lax.axis_index("c")


def _gather_body(cst_u, rst_u, lan_u, nu_u, cst_i, rst_i, lan_i, nu_i,
                 ufacT, ifacT, stg_u, stg_i,
                 col_v, rst_v, lan_v, nu_v, slab_v, stage_v, sem):
    wid = _wid()
    lane = lax.iota(jnp.int32, L)
    dvec = [lane + q * L for q in range(D // L)]

    for cst_h, rst_h, lan_h, nu_h, tbl, stg in (
            (cst_u, rst_u, lan_u, nu_u, ufacT, stg_u),
            (cst_i, rst_i, lan_i, nu_i, ifacT, stg_i)):
        pltpu.sync_copy(cst_h.at[wid], col_v)
        pltpu.sync_copy(rst_h.at[wid], rst_v)
        pltpu.sync_copy(lan_h.at[wid], lan_v)
        pltpu.sync_copy(nu_h, nu_v)
        nvec = nu_v[pl.ds((wid >> 4) << 4, L)]
        n = jnp.sum(jnp.where(lane == (wid & 15), nvec, 0))

        for pre in range(NBUF - 1):
            @pl.when(pre < n)
            def _prime(pre=pre):
                cp = pl.multiple_of(col_v[pl.ds(pre, L)][0], 128)
                pltpu.async_copy(tbl.at[:, pl.ds(cp, 128)],
                                 slab_v.at[pre], sem.at[pre])

        def slab_body(s, carry, tbl=tbl):
            @pl.when(s + NBUF - 1 < n)
            def _start_next():
                cn = pl.multiple_of(col_v[pl.ds(s + NBUF - 1, L)][0], 128)
                pltpu.async_copy(
                    tbl.at[:, pl.ds(cn, 128)],
                    slab_v.at[(s + NBUF - 1) & (NBUF - 1)],
                    sem.at[(s + NBUF - 1) & (NBUF - 1)])

            pltpu.make_async_copy(tbl.at[:, pl.ds(0, 128)],
                                  slab_v.at[s & (NBUF - 1)],
                                  sem.at[s & (NBUF - 1)]).wait()
            rvec = rst_v[pl.ds(s, L)]
            par = jnp.full((L,), s & (NBUF - 1), jnp.int32)

            def row_body(j, c2):
                l = lan_v[pl.ds(j, L)][0]
                lsp = jnp.full((L,), l, jnp.int32)
                for q in range(D // L):
                    g = plsc.load_gather(slab_v, [par, dvec[q], lsp])
                    stage_v[j >> 1, pl.ds((j & 1) * D + q * L, L)] = g
                return c2

            lax.fori_loop(rvec[0], rvec[1], row_body, 0)
            return carry

        lax.fori_loop(0, n, slab_body, 0)
        pltpu.sync_copy(stage_v, stg.at[pl.ds(wid * (BPW // 2), BPW // 2)])


def _pair_body(rk_u, rk_i, stg_u, stg_i, out_hbm,
               rku_v, rki_v, hu_v, hi_v, ub_v, ib_v, out_v, sem):
    wid = _wid()
    base = wid * BPW
    pltpu.sync_copy(rk_u.at[wid], rku_v)
    pltpu.sync_copy(rk_i.at[wid], rki_v)

    lane = lax.iota(jnp.int32, L)
    xor_idx = [lane ^ sh for sh in (8, 4, 2, 1)]
    zero = jnp.zeros((L,), jnp.float32)

    for blk in range(BPW // L):
        hu_v[pl.ds(blk * L, L)] = rku_v[pl.ds(blk * L, L)] >> 1
        hi_v[pl.ds(blk * L, L)] = rki_v[pl.ds(blk * L, L)] >> 1

    def _start(k):
        pltpu.async_copy(stg_u.at[hu_v.at[pl.ds(k * C, C)]],
                         ub_v.at[k & 1], sem.at[k & 1])
        pltpu.async_copy(stg_i.at[hi_v.at[pl.ds(k * C, C)]],
                         ib_v.at[k & 1], sem.at[k & 1])

    _start(0)

    nchunks = BPW // C

    def chunk_body(k, carry):
        @pl.when(k + 1 < nchunks)
        def _next():
            _start(k + 1)
        pltpu.make_async_copy(stg_u.at[pl.ds(0, C)], ub_v.at[k & 1],
                              sem.at[k & 1]).wait()
        pltpu.make_async_copy(stg_i.at[pl.ds(0, C)], ib_v.at[k & 1],
                              sem.at[k & 1]).wait()

        for blk in range(C // L):
            ruv = rku_v[pl.ds(k * C + blk * L, L)]
            riv = rki_v[pl.ds(k * C + blk * L, L)]
            acc = zero
            for j in range(L):
                slot = blk * L + j
                ou = (ruv[j] & 1) * D
                oi = (riv[j] & 1) * D
                p = (ub_v[k & 1, slot, pl.ds(ou, L)]
                     * ib_v[k & 1, slot, pl.ds(oi, L)])
                for q in range(1, D // L):
                    p += (ub_v[k & 1, slot, pl.ds(ou + q * L, L)]
                          * ib_v[k & 1, slot, pl.ds(oi + q * L, L)])
                for xi in xor_idx:
                    p = p + _take(p, xi)
                acc = jnp.where(lane == j, p, acc)
            out_v[pl.ds(k * C + blk * L, L)] = acc
        return carry

    lax.fori_loop(0, nchunks, chunk_body, 0)
    pltpu.sync_copy(out_v, out_hbm.at[pl.ds(base, BPW)])


def _side_routing(idx):
    iota = jnp.arange(idx.shape[0], dtype=jnp.int32)
    xs, order = lax.sort_key_val(idx, iota)
    _, rank = lax.sort_key_val(order, iota)
    cstart = (xs >> 7) << 7
    lanes = xs & 127
    c2 = cstart.reshape(NW, BPW)
    newflag = jnp.concatenate(
        [jnp.ones((NW, 1), jnp.bool_), c2[:, 1:] != c2[:, :-1]], axis=1)
    ord2 = jnp.cumsum(newflag.astype(jnp.int32), axis=1) - 1
    nu = (ord2[:, -1] + 1).astype(jnp.int32)
    karange = jnp.arange(IDXPAD, dtype=jnp.int32)[None, None, :]
    rst = (ord2[:, :, None] < karange).astype(jnp.int32).sum(axis=1)
    cst = jnp.take_along_axis(c2, jnp.minimum(rst, BPW - 1), axis=1,
                              mode="promise_in_bounds")
    lan = jnp.pad(lanes.reshape(NW, BPW), ((0, 0), (0, IDXPAD - BPW)))
    return cst, rst, lan, nu, rank.reshape(NW, BPW)


def kernel(user_item_tuple, user_factors, item_factors):
    batch = user_item_tuple.shape[0]
    uit = user_item_tuple.astype(jnp.int32)
    cst_u, rst_u, lan_u, nu_u, rk_u = _side_routing(uit[:, 0])
    cst_i, rst_i, lan_i, nu_i, rk_i = _side_routing(uit[:, 1])

    mesh = plsc.VectorSubcoreMesh(core_axis_name="c", subcore_axis_name="s")
    params = pltpu.CompilerParams(needs_layout_passes=False)
    stg_shape = jax.ShapeDtypeStruct((batch // 2, 2 * D), jnp.float32)

    gather_call = functools.partial(
        pl.kernel,
        out_type=(stg_shape, stg_shape),
        mesh=mesh,
        compiler_params=params,
        scratch_types=[
            pltpu.VMEM((IDXPAD,), jnp.int32),
            pltpu.VMEM((IDXPAD,), jnp.int32),
            pltpu.VMEM((IDXPAD,), jnp.int32),
            pltpu.VMEM((NW,), jnp.int32),
            pltpu.VMEM((NBUF, D, 128), jnp.float32),
            pltpu.VMEM((BPW // 2, 2 * D), jnp.float32),
            pltpu.SemaphoreType.DMA((NBUF,)),
        ],
    )(_gather_body)
    stg_u, stg_i = gather_call(
        cst_u, rst_u, lan_u, nu_u, cst_i, rst_i, lan_i, nu_i,
        user_factors.T, item_factors.T)

    pair_call = functools.partial(
        pl.kernel,
        out_type=jax.ShapeDtypeStruct((batch,), jnp.float32),
        mesh=mesh,
        compiler_params=params,
        scratch_types=[
            pltpu.VMEM((BPW,), jnp.int32),
            pltpu.VMEM((BPW,), jnp.int32),
            pltpu.VMEM((BPW,), jnp.int32),
            pltpu.VMEM((BPW,), jnp.int32),
            pltpu.VMEM((2, C, 2 * D), jnp.float32),
            pltpu.VMEM((2, C, 2 * D), jnp.float32),
            pltpu.VMEM((BPW,), jnp.float32),
            pltpu.SemaphoreType.DMA((2,)),
        ],
    )(_pair_body)
    return pair_call(rk_u, rk_i, stg_u, stg_i)


# speedup vs baseline: 2.6262x; 1.0248x over previous
import functools

import jax
import jax.numpy as jnp
from jax import lax
from jax.experimental import pallas as pl
from jax.experimental.pallas import tpu as pltpu
from jax.experimental.pallas import tpu_sc as plsc

D = 64
L = 16
NW = 32
BPW = 512
C = 128
NBUF = 8
IDXPAD = 528


def _take(v, idx):
    return jnp.take_along_axis(v, idx, axis=0, mode="promise_in_bounds")


def _wid():
    return lax.axis_index("s") * 2 + lax.axis_index("c")


def _gather_body(cst_u, rst_u, lan_u, nu_u, cst_i, rst_i, lan_i, nu_i,
                 ufacT, ifacT, stg_u, stg_i,
                 col_v, rst_v, lan_v, nu_v, slab_v, stage_v, sem):
    wid = _wid()
    lane = lax.iota(jnp.int32, L)
    dvec = [lane + q * L for q in range(D // L)]

    for cst_h, rst_h, lan_h, nu_h, tbl, stg in (
            (cst_u, rst_u, lan_u, nu_u, ufacT, stg_u),
            (cst_i, rst_i, lan_i, nu_i, ifacT, stg_i)):
        pltpu.sync_copy(cst_h.at[wid], col_v)
        pltpu.sync_copy(rst_h.at[wid], rst_v)
        pltpu.sync_copy(lan_h.at[wid], lan_v)
        pltpu.sync_copy(nu_h, nu_v)
        nvec = nu_v[pl.ds((wid >> 4) << 4, L)]
        n = jnp.sum(jnp.where(lane == (wid & 15), nvec, 0))

        for pre in range(NBUF - 1):
            @pl.when(pre < n)
            def _prime(pre=pre):
                cp = pl.multiple_of(col_v[pl.ds(pre, L)][0], 128)
                pltpu.async_copy(tbl.at[:, pl.ds(cp, 128)],
                                 slab_v.at[pre], sem.at[pre])

        def slab_body(s, carry, tbl=tbl):
            @pl.when(s + NBUF - 1 < n)
            def _start_next():
                cn = pl.multiple_of(col_v[pl.ds(s + NBUF - 1, L)][0], 128)
                pltpu.async_copy(
                    tbl.at[:, pl.ds(cn, 128)],
                    slab_v.at[(s + NBUF - 1) & (NBUF - 1)],
                    sem.at[(s + NBUF - 1) & (NBUF - 1)])

            pltpu.make_async_copy(tbl.at[:, pl.ds(0, 128)],
                                  slab_v.at[s & (NBUF - 1)],
                                  sem.at[s & (NBUF - 1)]).wait()
            rvec = rst_v[pl.ds(s, L)]
            par = jnp.full((L,), s & (NBUF - 1), jnp.int32)

            def row_body(j, c2):
                l = lan_v[pl.ds(j, L)][0]
                lsp = jnp.full((L,), l, jnp.int32)
                for q in range(D // L):
                    g = plsc.load_gather(slab_v, [par, dvec[q], lsp])
                    stage_v[j >> 1, pl.ds((j & 1) * D + q * L, L)] = g
                return c2

            lax.fori_loop(rvec[0], rvec[1], row_body, 0)
            return carry

        lax.fori_loop(0, n, slab_body, 0)
        pltpu.sync_copy(stage_v, stg.at[pl.ds(wid * (BPW // 2), BPW // 2)])


def _pair_body(rk_u, rk_i, stg_u, stg_i, out_hbm,
               rku_v, rki_v, hu_v, hi_v, ub_v, ib_v, out_v, sem):
    wid = _wid()
    base = wid * BPW
    pltpu.sync_copy(rk_u.at[wid], rku_v)
    pltpu.sync_copy(rk_i.at[wid], rki_v)

    lane = lax.iota(jnp.int32, L)
    xor_idx = [lane ^ sh for sh in (8, 4, 2, 1)]
    zero = jnp.zeros((L,), jnp.float32)

    for blk in range(BPW // L):
        hu_v[pl.ds(blk * L, L)] = rku_v[pl.ds(blk * L, L)] >> 1
        hi_v[pl.ds(blk * L, L)] = rki_v[pl.ds(blk * L, L)] >> 1

    def _start(k):
        pltpu.async_copy(stg_u.at[hu_v.at[pl.ds(k * C, C)]],
                         ub_v.at[k & 1], sem.at[k & 1])
        pltpu.async_copy(stg_i.at[hi_v.at[pl.ds(k * C, C)]],
                         ib_v.at[k & 1], sem.at[k & 1])

    _start(0)

    nchunks = BPW // C

    def chunk_body(k, carry):
        @pl.when(k + 1 < nchunks)
        def _next():
            _start(k + 1)
        pltpu.make_async_copy(stg_u.at[pl.ds(0, C)], ub_v.at[k & 1],
                              sem.at[k & 1]).wait()
        pltpu.make_async_copy(stg_i.at[pl.ds(0, C)], ib_v.at[k & 1],
                              sem.at[k & 1]).wait()

        for blk in range(C // L):
            ruv = rku_v[pl.ds(k * C + blk * L, L)]
            riv = rki_v[pl.ds(k * C + blk * L, L)]
            acc = zero
            for j in range(L):
                slot = blk * L + j
                ou = (ruv[j] & 1) * D
                oi = (riv[j] & 1) * D
                p = (ub_v[k & 1, slot, pl.ds(ou, L)]
                     * ib_v[k & 1, slot, pl.ds(oi, L)])
                for q in range(1, D // L):
                    p += (ub_v[k & 1, slot, pl.ds(ou + q * L, L)]
                          * ib_v[k & 1, slot, pl.ds(oi + q * L, L)])
                for xi in xor_idx:
                    p = p + _take(p, xi)
                acc = jnp.where(lane == j, p, acc)
            out_v[pl.ds(k * C + blk * L, L)] = acc
        return carry

    lax.fori_loop(0, nchunks, chunk_body, 0)
    pltpu.sync_copy(out_v, out_hbm.at[pl.ds(base, BPW)])


def _side_routing(idx):
    iota = jnp.arange(idx.shape[0], dtype=jnp.int32)
    xs, order = lax.sort_key_val(idx, iota)
    _, rank = lax.sort_key_val(order, iota)
    cstart = (xs >> 7) << 7
    lanes = xs & 127
    c2 = cstart.reshape(NW, BPW)
    newflag = jnp.concatenate(
        [jnp.ones((NW, 1), jnp.bool_), c2[:, 1:] != c2[:, :-1]], axis=1)
    ord2 = jnp.cumsum(newflag.astype(jnp.int32), axis=1) - 1
    nu = (ord2[:, -1] + 1).astype(jnp.int32)
    karange = jnp.arange(IDXPAD, dtype=jnp.int32)[None, None, :]
    rst = (ord2[:, :, None] < karange).astype(jnp.int32).sum(axis=1)
    cst = jnp.where(ord2[:, :, None] == karange, c2[:, :, None], 0
                    ).max(axis=1).astype(jnp.int32)
    lan = jnp.pad(lanes.reshape(NW, BPW), ((0, 0), (0, IDXPAD - BPW)))
    return cst, rst, lan, nu, rank.reshape(NW, BPW)


def kernel(user_item_tuple, user_factors, item_factors):
    batch = user_item_tuple.shape[0]
    uit = user_item_tuple.astype(jnp.int32)
    cst_u, rst_u, lan_u, nu_u, rk_u = _side_routing(uit[:, 0])
    cst_i, rst_i, lan_i, nu_i, rk_i = _side_routing(uit[:, 1])

    mesh = plsc.VectorSubcoreMesh(core_axis_name="c", subcore_axis_name="s")
    params = pltpu.CompilerParams(needs_layout_passes=False)
    stg_shape = jax.ShapeDtypeStruct((batch // 2, 2 * D), jnp.float32)

    gather_call = functools.partial(
        pl.kernel,
        out_type=(stg_shape, stg_shape),
        mesh=mesh,
        compiler_params=params,
        scratch_types=[
            pltpu.VMEM((IDXPAD,), jnp.int32),
            pltpu.VMEM((IDXPAD,), jnp.int32),
            pltpu.VMEM((IDXPAD,), jnp.int32),
            pltpu.VMEM((NW,), jnp.int32),
            pltpu.VMEM((NBUF, D, 128), jnp.float32),
            pltpu.VMEM((BPW // 2, 2 * D), jnp.float32),
            pltpu.SemaphoreType.DMA((NBUF,)),
        ],
    )(_gather_body)
    stg_u, stg_i = gather_call(
        cst_u, rst_u, lan_u, nu_u, cst_i, rst_i, lan_i, nu_i,
        user_factors.T, item_factors.T)

    pair_call = functools.partial(
        pl.kernel,
        out_type=jax.ShapeDtypeStruct((batch,), jnp.float32),
        mesh=mesh,
        compiler_params=params,
        scratch_types=[
            pltpu.VMEM((BPW,), jnp.int32),
            pltpu.VMEM((BPW,), jnp.int32),
            pltpu.VMEM((BPW,), jnp.int32),
            pltpu.VMEM((BPW,), jnp.int32),
            pltpu.VMEM((2, C, 2 * D), jnp.float32),
            pltpu.VMEM((2, C, 2 * D), jnp.float32),
            pltpu.VMEM((BPW,), jnp.float32),
            pltpu.SemaphoreType.DMA((2,)),
        ],
    )(_pair_body)
    return pair_call(rk_u, rk_i, stg_u, stg_i)


# speedup vs baseline: 2.7058x; 1.0303x over previous
import functools

import jax
import jax.numpy as jnp
from jax import lax
from jax.experimental import pallas as pl
from jax.experimental.pallas import tpu as pltpu
from jax.experimental.pallas import tpu_sc as plsc

D = 64
L = 16
NW = 32
BPW = 512
C = 128
NBUF = 8
IDXPAD = 528


def _take(v, idx):
    return jnp.take_along_axis(v, idx, axis=0, mode="promise_in_bounds")


def _wid():
    return lax.axis_index("s") * 2 + lax.axis_index("c")


def _gather_body(cst_h, rst_h, lan_h, nu_h, tbl, stg,
                 col_v, rst_v, lan_v, nu_v, slab_v, stage_v, sem):
    wid = _wid()
    lane = lax.iota(jnp.int32, L)
    dvec = [lane + q * L for q in range(D // L)]

    pltpu.sync_copy(cst_h.at[wid], col_v)
    pltpu.sync_copy(rst_h.at[wid], rst_v)
    pltpu.sync_copy(lan_h.at[wid], lan_v)
    pltpu.sync_copy(nu_h, nu_v)
    nvec = nu_v[pl.ds((wid >> 4) << 4, L)]
    n = jnp.sum(jnp.where(lane == (wid & 15), nvec, 0))

    for pre in range(NBUF - 1):
        @pl.when(pre < n)
        def _prime(pre=pre):
            cp = pl.multiple_of(col_v[pl.ds(pre, L)][0], 128)
            pltpu.async_copy(tbl.at[:, pl.ds(cp, 128)],
                             slab_v.at[pre], sem.at[pre])

    def slab_body(s, carry):
        @pl.when(s + NBUF - 1 < n)
        def _start_next():
            cn = pl.multiple_of(col_v[pl.ds(s + NBUF - 1, L)][0], 128)
            pltpu.async_copy(
                tbl.at[:, pl.ds(cn, 128)],
                slab_v.at[(s + NBUF - 1) & (NBUF - 1)],
                sem.at[(s + NBUF - 1) & (NBUF - 1)])

        pltpu.make_async_copy(tbl.at[:, pl.ds(0, 128)],
                              slab_v.at[s & (NBUF - 1)],
                              sem.at[s & (NBUF - 1)]).wait()
        rvec = rst_v[pl.ds(s, L)]
        par = jnp.full((L,), s & (NBUF - 1), jnp.int32)

        def row_body(j, c2):
            l = lan_v[pl.ds(j, L)][0]
            lsp = jnp.full((L,), l, jnp.int32)
            for q in range(D // L):
                g = plsc.load_gather(slab_v, [par, dvec[q], lsp])
                stage_v[j >> 1, pl.ds((j & 1) * D + q * L, L)] = g
            return c2

        lax.fori_loop(rvec[0], rvec[1], row_body, 0)
        return carry

    lax.fori_loop(0, n, slab_body, 0)
    pltpu.sync_copy(stage_v, stg.at[pl.ds(wid * (BPW // 2), BPW // 2)])


def _pair_body(rk_u, rk_i, stg_u, stg_i, out_hbm,
               rku_v, rki_v, hu_v, hi_v, ub_v, ib_v, out_v, sem):
    wid = _wid()
    base = wid * BPW
    pltpu.sync_copy(rk_u.at[wid], rku_v)
    pltpu.sync_copy(rk_i.at[wid], rki_v)

    lane = lax.iota(jnp.int32, L)
    xor_idx = [lane ^ sh for sh in (8, 4, 2, 1)]
    zero = jnp.zeros((L,), jnp.float32)

    for blk in range(BPW // L):
        hu_v[pl.ds(blk * L, L)] = rku_v[pl.ds(blk * L, L)] >> 1
        hi_v[pl.ds(blk * L, L)] = rki_v[pl.ds(blk * L, L)] >> 1

    def _start(k):
        pltpu.async_copy(stg_u.at[hu_v.at[pl.ds(k * C, C)]],
                         ub_v.at[k & 1], sem.at[k & 1])
        pltpu.async_copy(stg_i.at[hi_v.at[pl.ds(k * C, C)]],
                         ib_v.at[k & 1], sem.at[k & 1])

    _start(0)

    nchunks = BPW // C

    def chunk_body(k, carry):
        @pl.when(k + 1 < nchunks)
        def _next():
            _start(k + 1)
        pltpu.make_async_copy(stg_u.at[pl.ds(0, C)], ub_v.at[k & 1],
                              sem.at[k & 1]).wait()
        pltpu.make_async_copy(stg_i.at[pl.ds(0, C)], ib_v.at[k & 1],
                              sem.at[k & 1]).wait()

        for blk in range(C // L):
            ruv = rku_v[pl.ds(k * C + blk * L, L)]
            riv = rki_v[pl.ds(k * C + blk * L, L)]
            acc = zero
            for j in range(L):
                slot = blk * L + j
                ou = (ruv[j] & 1) * D
                oi = (riv[j] & 1) * D
                p = (ub_v[k & 1, slot, pl.ds(ou, L)]
                     * ib_v[k & 1, slot, pl.ds(oi, L)])
                for q in range(1, D // L):
                    p += (ub_v[k & 1, slot, pl.ds(ou + q * L, L)]
                          * ib_v[k & 1, slot, pl.ds(oi + q * L, L)])
                for xi in xor_idx:
                    p = p + _take(p, xi)
                acc = jnp.where(lane == j, p, acc)
            out_v[pl.ds(k * C + blk * L, L)] = acc
        return carry

    lax.fori_loop(0, nchunks, chunk_body, 0)
    pltpu.sync_copy(out_v, out_hbm.at[pl.ds(base, BPW)])


def _side_routing(idx):
    iota = jnp.arange(idx.shape[0], dtype=jnp.int32)
    xs, order = lax.sort_key_val(idx, iota)
    _, rank = lax.sort_key_val(order, iota)
    cstart = (xs >> 7) << 7
    lanes = xs & 127
    c2 = cstart.reshape(NW, BPW)
    newflag = jnp.concatenate(
        [jnp.ones((NW, 1), jnp.bool_), c2[:, 1:] != c2[:, :-1]], axis=1)
    ord2 = jnp.cumsum(newflag.astype(jnp.int32), axis=1) - 1
    nu = (ord2[:, -1] + 1).astype(jnp.int32)
    karange = jnp.arange(IDXPAD, dtype=jnp.int32)[None, None, :]
    rst = (ord2[:, :, None] < karange).astype(jnp.int32).sum(axis=1)
    cst = jnp.where(ord2[:, :, None] == karange, c2[:, :, None], 0
                    ).max(axis=1).astype(jnp.int32)
    lan = jnp.pad(lanes.reshape(NW, BPW), ((0, 0), (0, IDXPAD - BPW)))
    return cst, rst, lan, nu, rank.reshape(NW, BPW)


def kernel(user_item_tuple, user_factors, item_factors):
    batch = user_item_tuple.shape[0]
    uit = user_item_tuple.astype(jnp.int32)
    cst_u, rst_u, lan_u, nu_u, rk_u = _side_routing(uit[:, 0])
    cst_i, rst_i, lan_i, nu_i, rk_i = _side_routing(uit[:, 1])

    mesh = plsc.VectorSubcoreMesh(core_axis_name="c", subcore_axis_name="s")
    params = pltpu.CompilerParams(needs_layout_passes=False)
    stg_shape = jax.ShapeDtypeStruct((batch // 2, 2 * D), jnp.float32)

    gather_call = functools.partial(
        pl.kernel,
        out_type=stg_shape,
        mesh=mesh,
        compiler_params=params,
        scratch_types=[
            pltpu.VMEM((IDXPAD,), jnp.int32),
            pltpu.VMEM((IDXPAD,), jnp.int32),
            pltpu.VMEM((IDXPAD,), jnp.int32),
            pltpu.VMEM((NW,), jnp.int32),
            pltpu.VMEM((NBUF, D, 128), jnp.float32),
            pltpu.VMEM((BPW // 2, 2 * D), jnp.float32),
            pltpu.SemaphoreType.DMA((NBUF,)),
        ],
    )(_gather_body)
    stg_u = gather_call(cst_u, rst_u, lan_u, nu_u, user_factors.T)
    stg_i = gather_call(cst_i, rst_i, lan_i, nu_i, item_factors.T)

    pair_call = functools.partial(
        pl.kernel,
        out_type=jax.ShapeDtypeStruct((batch,), jnp.float32),
        mesh=mesh,
        compiler_params=params,
        scratch_types=[
            pltpu.VMEM((BPW,), jnp.int32),
            pltpu.VMEM((BPW,), jnp.int32),
            pltpu.VMEM((BPW,), jnp.int32),
            pltpu.VMEM((BPW,), jnp.int32),
            pltpu.VMEM((2, C, 2 * D), jnp.float32),
            pltpu.VMEM((2, C, 2 * D), jnp.float32),
            pltpu.VMEM((BPW,), jnp.float32),
            pltpu.SemaphoreType.DMA((2,)),
        ],
    )(_pair_body)
    return pair_call(rk_u, rk_i, stg_u, stg_i)


# speedup vs baseline: 2.7741x; 1.0252x over previous
import functools

import jax
import jax.numpy as jnp
from jax import lax
from jax.experimental import pallas as pl
from jax.experimental.pallas import tpu as pltpu
from jax.experimental.pallas import tpu_sc as plsc

D = 64
L = 16
NW = 32
BPW = 512
C = 128
NBUF = 8
IDXPAD = 528


def _take(v, idx):
    return jnp.take_along_axis(v, idx, axis=0, mode="promise_in_bounds")


def _wid():
    return lax.axis_index("s") * 2 + lax.axis_index("c")


def _gather_body(cst_h, rst_h, lan_h, nu_h, tbl, stg,
                 col_v, rst_v, lan_v, nu_v, slab_v, stage_v, sem):
    wid = _wid()
    lane = lax.iota(jnp.int32, L)
    dvec = [lane + q * L for q in range(D // L)]

    pltpu.sync_copy(cst_h.at[wid], col_v)
    pltpu.sync_copy(rst_h.at[wid], rst_v)
    pltpu.sync_copy(lan_h.at[wid], lan_v)
    pltpu.sync_copy(nu_h, nu_v)
    nvec = nu_v[pl.ds((wid >> 4) << 4, L)]
    n = jnp.sum(jnp.where(lane == (wid & 15), nvec, 0))

    for pre in range(NBUF - 1):
        @pl.when(pre < n)
        def _prime(pre=pre):
            cp = pl.multiple_of(col_v[pl.ds(pre, L)][0], 128)
            pltpu.async_copy(tbl.at[:, pl.ds(cp, 128)],
                             slab_v.at[pre], sem.at[pre])

    def slab_body(s, carry):
        @pl.when(s + NBUF - 1 < n)
        def _start_next():
            cn = pl.multiple_of(col_v[pl.ds(s + NBUF - 1, L)][0], 128)
            pltpu.async_copy(
                tbl.at[:, pl.ds(cn, 128)],
                slab_v.at[(s + NBUF - 1) & (NBUF - 1)],
                sem.at[(s + NBUF - 1) & (NBUF - 1)])

        pltpu.make_async_copy(tbl.at[:, pl.ds(0, 128)],
                              slab_v.at[s & (NBUF - 1)],
                              sem.at[s & (NBUF - 1)]).wait()
        rvec = rst_v[pl.ds(s, L)]
        par = jnp.full((L,), s & (NBUF - 1), jnp.int32)

        def row_body(j, c2):
            l = lan_v[pl.ds(j, L)][0]
            lsp = jnp.full((L,), l, jnp.int32)
            for q in range(D // L):
                g = plsc.load_gather(slab_v, [par, dvec[q], lsp])
                stage_v[j >> 1, pl.ds((j & 1) * D + q * L, L)] = g
            return c2

        lax.fori_loop(rvec[0], rvec[1], row_body, 0)
        return carry

    lax.fori_loop(0, n, slab_body, 0)
    pltpu.sync_copy(stage_v, stg.at[pl.ds(wid * (BPW // 2), BPW // 2)])


def _pair_body(rk_u, rk_i, stg_u, stg_i, out_hbm,
               rku_v, rki_v, hu_v, hi_v, ub_v, ib_v, out_v, sem):
    wid = _wid()
    base = wid * BPW
    pltpu.sync_copy(rk_u.at[wid], rku_v)
    pltpu.sync_copy(rk_i.at[wid], rki_v)

    lane = lax.iota(jnp.int32, L)
    xor_idx = [lane ^ sh for sh in (8, 4, 2, 1)]
    zero = jnp.zeros((L,), jnp.float32)

    for blk in range(BPW // L):
        hu_v[pl.ds(blk * L, L)] = rku_v[pl.ds(blk * L, L)] >> 1
        hi_v[pl.ds(blk * L, L)] = rki_v[pl.ds(blk * L, L)] >> 1

    def _start(k):
        pltpu.async_copy(stg_u.at[hu_v.at[pl.ds(k * C, C)]],
                         ub_v.at[k & 1], sem.at[k & 1])
        pltpu.async_copy(stg_i.at[hi_v.at[pl.ds(k * C, C)]],
                         ib_v.at[k & 1], sem.at[k & 1])

    _start(0)

    nchunks = BPW // C

    def chunk_body(k, carry):
        @pl.when(k + 1 < nchunks)
        def _next():
            _start(k + 1)
        pltpu.make_async_copy(stg_u.at[pl.ds(0, C)], ub_v.at[k & 1],
                              sem.at[k & 1]).wait()
        pltpu.make_async_copy(stg_i.at[pl.ds(0, C)], ib_v.at[k & 1],
                              sem.at[k & 1]).wait()

        for blk in range(C // L):
            ruv = rku_v[pl.ds(k * C + blk * L, L)]
            riv = rki_v[pl.ds(k * C + blk * L, L)]
            acc = zero
            for j in range(L):
                slot = blk * L + j
                ou = (ruv[j] & 1) * D
                oi = (riv[j] & 1) * D
                p = (ub_v[k & 1, slot, pl.ds(ou, L)]
                     * ib_v[k & 1, slot, pl.ds(oi, L)])
                for q in range(1, D // L):
                    p += (ub_v[k & 1, slot, pl.ds(ou + q * L, L)]
                          * ib_v[k & 1, slot, pl.ds(oi + q * L, L)])
                for xi in xor_idx:
                    p = p + _take(p, xi)
                acc = jnp.where(lane == j, p, acc)
            out_v[pl.ds(k * C + blk * L, L)] = acc
        return carry

    lax.fori_loop(0, nchunks, chunk_body, 0)
    pltpu.sync_copy(out_v, out_hbm.at[pl.ds(base, BPW)])


def _side_routing(idx):
    iota = jnp.arange(idx.shape[0], dtype=jnp.int32)
    xs, order = lax.sort_key_val(idx, iota)
    _, rank = lax.sort_key_val(order, iota)
    cstart = (xs >> 7) << 7
    lanes = xs & 127
    c2 = cstart.reshape(NW, BPW)
    newflag = jnp.concatenate(
        [jnp.ones((NW, 1), jnp.bool_), c2[:, 1:] != c2[:, :-1]], axis=1)
    nu = newflag.sum(axis=1).astype(jnp.int32)
    jr = jnp.broadcast_to(
        jnp.arange(BPW, dtype=jnp.int32)[None, :], (NW, BPW))
    key = jnp.where(newflag, jr, BPW)
    rst2, cst2 = lax.sort_key_val(key, c2)
    pad = ((0, 0), (0, IDXPAD - BPW))
    rst = jnp.pad(rst2, pad, constant_values=BPW)
    cst = jnp.pad(cst2, pad)
    lan = jnp.pad(lanes.reshape(NW, BPW), pad)
    return cst, rst, lan, nu, rank.reshape(NW, BPW)


def kernel(user_item_tuple, user_factors, item_factors):
    batch = user_item_tuple.shape[0]
    uit = user_item_tuple.astype(jnp.int32)
    cst_u, rst_u, lan_u, nu_u, rk_u = _side_routing(uit[:, 0])
    cst_i, rst_i, lan_i, nu_i, rk_i = _side_routing(uit[:, 1])

    mesh = plsc.VectorSubcoreMesh(core_axis_name="c", subcore_axis_name="s")
    params = pltpu.CompilerParams(needs_layout_passes=False)
    stg_shape = jax.ShapeDtypeStruct((batch // 2, 2 * D), jnp.float32)

    gather_call = functools.partial(
        pl.kernel,
        out_type=stg_shape,
        mesh=mesh,
        compiler_params=params,
        scratch_types=[
            pltpu.VMEM((IDXPAD,), jnp.int32),
            pltpu.VMEM((IDXPAD,), jnp.int32),
            pltpu.VMEM((IDXPAD,), jnp.int32),
            pltpu.VMEM((NW,), jnp.int32),
            pltpu.VMEM((NBUF, D, 128), jnp.float32),
            pltpu.VMEM((BPW // 2, 2 * D), jnp.float32),
            pltpu.SemaphoreType.DMA((NBUF,)),
        ],
    )(_gather_body)
    stg_u = gather_call(cst_u, rst_u, lan_u, nu_u, user_factors.T)
    stg_i = gather_call(cst_i, rst_i, lan_i, nu_i, item_factors.T)

    pair_call = functools.partial(
        pl.kernel,
        out_type=jax.ShapeDtypeStruct((batch,), jnp.float32),
        mesh=mesh,
        compiler_params=params,
        scratch_types=[
            pltpu.VMEM((BPW,), jnp.int32),
            pltpu.VMEM((BPW,), jnp.int32),
            pltpu.VMEM((BPW,), jnp.int32),
            pltpu.VMEM((BPW,), jnp.int32),
            pltpu.VMEM((2, C, 2 * D), jnp.float32),
            pltpu.VMEM((2, C, 2 * D), jnp.float32),
            pltpu.VMEM((BPW,), jnp.float32),
            pltpu.SemaphoreType.DMA((2,)),
        ],
    )(_pair_body)
    return pair_call(rk_u, rk_i, stg_u, stg_i)
